# baseline (speedup 1.0000x reference)
import functools

import jax
import jax.numpy as jnp
from jax import lax
from jax.experimental import pallas as pl
from jax.experimental.pallas import tpu as pltpu
from jax.experimental.pallas import tpu_sc as plsc


@functools.lru_cache(maxsize=None)
def _build(samples: int, time: int, dim: int):
    nc = 1
    nchunks = 4
    rows = dim // nc
    crows = rows // nchunks
    assert rows % nchunks == 0 and crows % 8 == 0

    mesh = plsc.ScalarSubcoreMesh(axis_name="c", num_cores=nc)

    @functools.partial(
        pl.kernel,
        out_type=jax.ShapeDtypeStruct((samples, dim, time), jnp.float32),
        mesh=mesh,
        scratch_types=[
            pltpu.VMEM_SHARED((rows, time), jnp.float32),
            pltpu.SemaphoreType.DMA,
            pltpu.SemaphoreType.DMA,
        ],
        compiler_params=pltpu.CompilerParams(use_tc_tiling_on_sc=True),
    )
    def tile_kernel(table_hbm, out_hbm, spbuf, in_sem, out_sem):
        base = lax.axis_index("c") * rows
        loads = [
            pltpu.async_copy(
                table_hbm.at[pl.ds(base + k * crows, crows), :],
                spbuf.at[pl.ds(k * crows, crows), :],
                in_sem,
            )
            for k in range(nchunks)
        ]
        stores = []
        for k in range(nchunks):
            loads[k].wait()
            stores += [
                pltpu.async_copy(
                    spbuf.at[pl.ds(k * crows, crows), :],
                    out_hbm.at[s, pl.ds(base + k * crows, crows), :],
                    out_sem,
                )
                for s in range(samples)
            ]
        for c in stores:
            c.wait()

    return tile_kernel


def kernel(x, table):
    table_t = jnp.swapaxes(table, 0, 1)
    out_t = _build(x.shape[0], table.shape[0], table.shape[1])(table_t)
    return jnp.swapaxes(out_t, 1, 2)
